# baseline (speedup 1.0000x reference)
import functools

import jax
import jax.numpy as jnp
from jax import lax
from jax.experimental import pallas as pl
from jax.experimental.pallas import tpu as pltpu
from jax.experimental.pallas import tpu_sc as plsc

N = 10000
C = 128
K = 16
KD = 32
E = N * K

NTILES = 32
NP = 320
NPAD = NTILES * NP
CH = 4000
NCH = E // CH
GB = 128
LANES = 16

ROW_BLK = 1000
GRID = N // ROW_BLK

_NEG_INF = float("-inf")


OB = CH + GB + LANES


def _segmax_body(src_hbm, dst_hbm, b_hbm, out_hbm,
                 acc, dbuf, sbuf, osrc, odst, rows, scr, smem,
                 sem_d, sem_s, sem_g):
  cid = lax.axis_index("c")
  sid = lax.axis_index("s")
  wid = sid * 2 + cid
  lo = wid * NP
  ii = lax.iota(jnp.int32, LANES)

  neg = jnp.full((LANES,), _NEG_INF, jnp.float32)
  zero16 = jnp.zeros((LANES,), jnp.int32)
  sent = jnp.full((LANES,), NP, jnp.int32)

  def init_acc(i, _):
    r = i // (C // LANES)
    c = i % (C // LANES)
    acc[r, pl.ds(c * LANES, LANES)] = neg
    return 0
  lax.fori_loop(0, (NP + 1) * (C // LANES), init_acc, 0)

  scr[pl.ds(0, LANES)] = zero16
  scr[pl.ds(LANES, LANES)] = zero16

  def init_idx(i, _):
    osrc[pl.ds(i * LANES, LANES)] = zero16
    odst[pl.ds(i * LANES, LANES)] = sent
    return 0
  lax.fori_loop(0, OB // LANES, init_idx, 0)

  def chunk_body(ci, _):
    off = ci * CH
    cp_d = pltpu.async_copy(dst_hbm.at[pl.ds(off, CH)], dbuf, sem_d)
    cp_s = pltpu.async_copy(src_hbm.at[pl.ds(off, CH)], sbuf, sem_s)
    cp_d.wait()
    cp_s.wait()

    smem[0] = 0

    def comp_body(g, _):
      vd = dbuf[pl.ds(g * LANES, LANES)]
      vs = sbuf[pl.ds(g * LANES, LANES)]
      u = vd - lo
      m = (u >= 0) & (u < NP)
      x = jnp.where(m, jnp.int32(1) << ii, 0)
      for k in (1, 2, 4, 8):
        scr[pl.ds(LANES, LANES)] = x
        x = x + scr[pl.ds(LANES - k, LANES)]
      w0 = x[LANES - 1]

      def pop_chain(w, depth):
        @pl.when(w != 0)
        def _():
          oc = smem[0]
          lsb = w & (-w)
          fb = lax.bitcast_convert_type(lsb.astype(jnp.float32), jnp.int32)
          lv = lax.broadcast((fb >> 23) - 127, (LANES,))
          osrc[pl.ds(oc, LANES)] = vs.at[lv].get(mode="promise_in_bounds")
          odst[pl.ds(oc, LANES)] = u.at[lv].get(mode="promise_in_bounds")
          smem[0] = oc + 1
          if depth + 1 < LANES:
            pop_chain(w & (w - 1), depth + 1)

      pop_chain(w0, 0)
      return 0
    lax.fori_loop(0, CH // LANES, comp_body, 0)

    cnt = smem[0]
    for i in range(GB // LANES + 1):
      odst[pl.ds(cnt + i * LANES, LANES)] = sent

    def batch_body(bi, _):
      bo = bi * GB
      pltpu.async_copy(b_hbm.at[osrc.at[pl.ds(bo, GB)]], rows, sem_g).wait()

      def group_body(g, _):
        dv = odst[pl.ds(bo + g * LANES, LANES)]
        for lane in range(LANES):
          d = dv[lane]
          e = g * LANES + lane
          for c in range(C // LANES):
            sl = pl.ds(c * LANES, LANES)
            acc[d, sl] = jnp.maximum(acc[d, sl], rows[e, sl])
        return 0
      lax.fori_loop(0, GB // LANES, group_body, 0)
      return 0
    nbatch = (cnt + (GB - 1)) // GB
    lax.fori_loop(0, nbatch, batch_body, 0)
    return 0

  lax.fori_loop(0, NCH, chunk_body, 0)

  pltpu.sync_copy(acc.at[pl.ds(0, NP)], out_hbm.at[pl.ds(lo, NP)])


@jax.jit
def _segmax(src, dst, b_mat):
  mesh = plsc.VectorSubcoreMesh(core_axis_name="c", subcore_axis_name="s")
  return pl.kernel(
      _segmax_body,
      out_type=jax.ShapeDtypeStruct((NPAD, C), jnp.float32),
      mesh=mesh,
      scratch_types=[
          pltpu.VMEM((NP + 1, C), jnp.float32),
          pltpu.VMEM((CH,), jnp.int32),
          pltpu.VMEM((CH,), jnp.int32),
          pltpu.VMEM((OB,), jnp.int32),
          pltpu.VMEM((OB,), jnp.int32),
          pltpu.VMEM((GB, C), jnp.float32),
          pltpu.VMEM((2 * LANES,), jnp.int32),
          pltpu.SMEM((8,), jnp.int32),
          pltpu.SemaphoreType.DMA,
          pltpu.SemaphoreType.DMA,
          pltpu.SemaphoreType.DMA,
      ],
  )(src, dst, b_mat)


def _leaky(v):
  return jnp.where(v >= 0, v, 0.2 * v)


def _h_from(a_ref, s_ref):
  s = s_ref[...]
  return jnp.where(jnp.isfinite(s), _leaky(a_ref[...] + s), 0.0)


def _tc1_body(x_ref, w0_ref, bb0_ref, w1_ref, bb1_ref,
              a0_ref, b0_ref, a1_ref, b1_ref):
  x = x_ref[...]
  for w_ref, bias_ref, a_out, b_out in (
      (w0_ref, bb0_ref, a0_ref, b0_ref),
      (w1_ref, bb1_ref, a1_ref, b1_ref),
  ):
    w = w_ref[...]
    wi, wj = w[:C], w[C:]
    b_out[...] = jnp.dot(x, wj, preferred_element_type=jnp.float32)
    a_out[...] = (jnp.dot(x, wi - wj, preferred_element_type=jnp.float32)
                  + bias_ref[...])


def _tc2_body(x_ref, a00_ref, s00_ref, w0_ref, bb0_ref,
              a01_ref, s01_ref, w1_ref, bb1_ref,
              h00_ref, a10_ref, b10_ref, h01_ref, a11_ref, b11_ref):
  x = x_ref[...]
  for a0_ref, s0_ref, w_ref, bias_ref, h_out, a_out, b_out in (
      (a00_ref, s00_ref, w0_ref, bb0_ref, h00_ref, a10_ref, b10_ref),
      (a01_ref, s01_ref, w1_ref, bb1_ref, h01_ref, a11_ref, b11_ref),
  ):
    h0 = _h_from(a0_ref, s0_ref)
    h_out[...] = h0
    w = w_ref[...]
    wi, wj = w[: 2 * C], w[2 * C :]
    wd = wi - wj
    b_out[...] = (jnp.dot(x, wj[:C], preferred_element_type=jnp.float32)
                  + jnp.dot(h0, wj[C:], preferred_element_type=jnp.float32))
    a_out[...] = (jnp.dot(x, wd[:C], preferred_element_type=jnp.float32)
                  + jnp.dot(h0, wd[C:], preferred_element_type=jnp.float32)
                  + bias_ref[...])


def _tc3_body(x_ref, h00_ref, a10_ref, s10_ref, wf0_ref, bf0_ref,
              h01_ref, a11_ref, s11_ref, wf1_ref, bf1_ref, out_ref):
  x = x_ref[...]
  outs = []
  for h0_ref, a1_ref, s1_ref, wf_ref, bf_ref in (
      (h00_ref, a10_ref, s10_ref, wf0_ref, bf0_ref),
      (h01_ref, a11_ref, s11_ref, wf1_ref, bf1_ref),
  ):
    h1 = _h_from(a1_ref, s1_ref)
    wf = wf_ref[...]
    o = (jnp.dot(x, wf[:C], preferred_element_type=jnp.float32)
         + jnp.dot(h0_ref[...], wf[C : 2 * C],
                   preferred_element_type=jnp.float32)
         + jnp.dot(h1, wf[2 * C :], preferred_element_type=jnp.float32)
         + bf_ref[...])
    outs.append(o)
  out_ref[...] = jnp.maximum(outs[0], outs[1]) + x


def _row_spec():
  return pl.BlockSpec((ROW_BLK, C), lambda i: (i, 0))


def _full_spec(shape):
  return pl.BlockSpec(shape, lambda i: tuple(0 for _ in shape))


def _nc():
  return jax.ShapeDtypeStruct((N, C), jnp.float32)


@jax.jit
def _tc1(x, w0, bb0, w1, bb1):
  return pl.pallas_call(
      _tc1_body,
      grid=(GRID,),
      in_specs=[
          _row_spec(),
          _full_spec((2 * C, C)), _full_spec((1, C)),
          _full_spec((2 * C, C)), _full_spec((1, C)),
      ],
      out_specs=[_row_spec()] * 4,
      out_shape=[_nc()] * 4,
  )(x, w0, bb0, w1, bb1)


@jax.jit
def _tc2(x, a00, s00, w0, bb0, a01, s01, w1, bb1):
  return pl.pallas_call(
      _tc2_body,
      grid=(GRID,),
      in_specs=[
          _row_spec(),
          _row_spec(), _row_spec(), _full_spec((4 * C, C)), _full_spec((1, C)),
          _row_spec(), _row_spec(), _full_spec((4 * C, C)), _full_spec((1, C)),
      ],
      out_specs=[_row_spec()] * 6,
      out_shape=[_nc()] * 6,
  )(x, a00, s00, w0, bb0, a01, s01, w1, bb1)


@jax.jit
def _tc3(x, h00, a10, s10, wf0, bf0, h01, a11, s11, wf1, bf1):
  return pl.pallas_call(
      _tc3_body,
      grid=(GRID,),
      in_specs=[
          _row_spec(),
          _row_spec(), _row_spec(), _row_spec(),
          _full_spec((3 * C, C)), _full_spec((1, C)),
          _row_spec(), _row_spec(), _row_spec(),
          _full_spec((3 * C, C)), _full_spec((1, C)),
      ],
      out_specs=_row_spec(),
      out_shape=_nc(),
  )(x, h00, a10, s10, wf0, bf0, h01, a11, s11, wf1, bf1)


def kernel(x, edge_index, W0b0, b0b0, W0b1, b0b1, W0f, b0f,
           W1b0, b1b0, W1b1, b1b1, W1f, b1f):
  ei = edge_index.reshape(2, N, KD)
  src1 = ei[0, :, 0:K].reshape(-1)
  dst1 = ei[1, :, 0:K].reshape(-1)
  src2 = ei[0, :, 0 : 2 * K : 2].reshape(-1)
  dst2 = ei[1, :, 0 : 2 * K : 2].reshape(-1)

  b0b0r = b0b0.reshape(1, C)
  b1b0r = b1b0.reshape(1, C)
  b0b1r = b0b1.reshape(1, C)
  b1b1r = b1b1.reshape(1, C)
  b0fr = b0f.reshape(1, C)
  b1fr = b1f.reshape(1, C)

  a00, b00, a01, b01 = _tc1(x, W0b0, b0b0r, W1b0, b1b0r)
  s00 = _segmax(src1, dst1, b00)[:N]
  s01 = _segmax(src2, dst2, b01)[:N]
  h00, a10, b10, h01, a11, b11 = _tc2(
      x, a00, s00, W0b1, b0b1r, a01, s01, W1b1, b1b1r)
  s10 = _segmax(src1, dst1, b10)[:N]
  s11 = _segmax(src2, dst2, b11)[:N]
  return _tc3(x, h00, a10, s10, W0f, b0fr, h01, a11, s11, W1f, b1fr)
